# Initial kernel scaffold; baseline (speedup 1.0000x reference)
import jax
import jax.numpy as jnp
from jax.experimental import pallas as pl


def kernel(x, edge_index):
    raise NotImplementedError("write your pallas kernel here")


# speedup vs baseline: 7.9346x; 7.9346x over previous
import functools

import jax
import jax.numpy as jnp
from jax import lax
from jax.experimental import pallas as pl
from jax.experimental.pallas import tpu as pltpu
from jax.experimental.pallas import tpu_sc as plsc

N_NODES = 10000
D_FEAT = 128
N_EDGES = 320000

NC = 2
NS = 16
NW = NC * NS
EDGES_PER_W = N_EDGES // NW
CHUNK = 80
NCHUNK = EDGES_PER_W // CHUNK
ZROWS = 125
ROWS_PER_TILE = N_NODES // NS
LANES = 16

_mesh = plsc.VectorSubcoreMesh(core_axis_name="c", subcore_axis_name="s")


@functools.partial(
    pl.kernel,
    out_type=jax.ShapeDtypeStruct((NC, N_NODES, D_FEAT), jnp.float32),
    mesh=_mesh,
    compiler_params=pltpu.CompilerParams(use_tc_tiling_on_sc=False),
    scratch_types=[
        pltpu.VMEM((NCHUNK, CHUNK), jnp.int32),
        pltpu.VMEM((NCHUNK, CHUNK), jnp.int32),
        pltpu.VMEM((CHUNK, D_FEAT), jnp.float32),
        pltpu.VMEM((ZROWS, D_FEAT), jnp.float32),
        pltpu.VMEM_SHARED((N_NODES, D_FEAT), jnp.float32),
        pltpu.SemaphoreType.DMA,
    ],
)
def _gather_scatter_sc(x_hbm, row_hbm, col_hbm, out_hbm,
                       row_v, col_v, xbuf, zbuf, acc, sem):
    c = lax.axis_index("c")
    s = lax.axis_index("s")
    wid = s * NC + c

    pltpu.sync_copy(row_hbm.at[wid], row_v)
    pltpu.sync_copy(col_hbm.at[wid], col_v)

    zv = jnp.zeros((LANES,), jnp.float32)

    def _zrow(i, _):
        def _zcol(k, _):
            zbuf[i, pl.ds(k * LANES, LANES)] = zv
            return 0
        return lax.fori_loop(0, D_FEAT // LANES, _zcol, 0)

    lax.fori_loop(0, ZROWS, _zrow, 0)

    base_row = s * ROWS_PER_TILE

    def _zacc(j, _):
        pltpu.sync_copy(zbuf, acc.at[pl.ds(base_row + j * ZROWS, ZROWS)])
        return 0

    lax.fori_loop(0, ROWS_PER_TILE // ZROWS, _zacc, 0)
    plsc.subcore_barrier()

    def _step(j, _):
        pltpu.async_copy(x_hbm.at[row_v.at[j]], xbuf, sem).wait()
        pltpu.sync_copy(xbuf, acc.at[col_v.at[j]], add=True)
        return 0

    lax.fori_loop(0, NCHUNK, _step, 0)
    plsc.subcore_barrier()

    def _wout(j, _):
        r0 = base_row + j * ZROWS
        pltpu.sync_copy(acc.at[pl.ds(r0, ZROWS)], out_hbm.at[c, pl.ds(r0, ZROWS)])
        return 0

    lax.fori_loop(0, ROWS_PER_TILE // ZROWS, _wout, 0)


def _combine_body(p_ref, o_ref):
    o_ref[...] = p_ref[0] + p_ref[1]


_combine_tc = pl.pallas_call(
    _combine_body,
    grid=(10,),
    in_specs=[pl.BlockSpec((2, N_NODES // 10, D_FEAT), lambda i: (0, i, 0))],
    out_specs=pl.BlockSpec((N_NODES // 10, D_FEAT), lambda i: (i, 0)),
    out_shape=jax.ShapeDtypeStruct((N_NODES, D_FEAT), jnp.float32),
)


def kernel(x, edge_index):
    row = edge_index[0].astype(jnp.int32).reshape(NW, NCHUNK, CHUNK)
    col = edge_index[1].astype(jnp.int32).reshape(NW, NCHUNK, CHUNK)
    partials = _gather_scatter_sc(x, row, col)
    return _combine_tc(partials)


# speedup vs baseline: 12.4057x; 1.5635x over previous
import functools

import jax
import jax.numpy as jnp
from jax import lax
from jax.experimental import pallas as pl
from jax.experimental.pallas import tpu as pltpu
from jax.experimental.pallas import tpu_sc as plsc

N_NODES = 10000
D_FEAT = 128
N_EDGES = 320000

NC = 2
NS = 16
NW = NC * NS
EDGES_PER_W = N_EDGES // NW
CHUNK = 80
NCHUNK = EDGES_PER_W // CHUNK
NPAIR = (NCHUNK - 1) // 2
ZROWS = 25
ROWS_PER_TILE = N_NODES // NS
WROWS = 125
LANES = 16

_mesh = plsc.VectorSubcoreMesh(core_axis_name="c", subcore_axis_name="s")


@functools.partial(
    pl.kernel,
    out_type=jax.ShapeDtypeStruct((NC, N_NODES, D_FEAT), jnp.float32),
    mesh=_mesh,
    compiler_params=pltpu.CompilerParams(use_tc_tiling_on_sc=False),
    scratch_types=[
        pltpu.VMEM((NCHUNK, CHUNK), jnp.int32),
        pltpu.VMEM((NCHUNK, CHUNK), jnp.int32),
        pltpu.VMEM((CHUNK, D_FEAT), jnp.float32),
        pltpu.VMEM((CHUNK, D_FEAT), jnp.float32),
        pltpu.VMEM_SHARED((N_NODES, D_FEAT), jnp.float32),
        pltpu.SemaphoreType.DMA,
        pltpu.SemaphoreType.DMA,
    ],
)
def _gather_scatter_sc(x_hbm, row_hbm, col_hbm, out_hbm,
                       row_v, col_v, xa, xb, acc, sema, semb):
    c = lax.axis_index("c")
    s = lax.axis_index("s")
    wid = s * NC + c

    pltpu.sync_copy(row_hbm.at[wid], row_v)
    pltpu.sync_copy(col_hbm.at[wid], col_v)

    zv = jnp.zeros((LANES,), jnp.float32)

    def _zrow(i, _):
        def _zcol(k, _):
            xa[i, pl.ds(k * LANES, LANES)] = zv
            return 0
        return lax.fori_loop(0, D_FEAT // LANES, _zcol, 0)

    lax.fori_loop(0, ZROWS, _zrow, 0)

    base_row = s * ROWS_PER_TILE
    zsrc = xa.at[pl.ds(0, ZROWS)]

    def _zacc(j, _):
        pltpu.sync_copy(zsrc, acc.at[pl.ds(base_row + j * ZROWS, ZROWS)])
        return 0

    lax.fori_loop(0, ROWS_PER_TILE // ZROWS, _zacc, 0)
    plsc.subcore_barrier()

    pltpu.async_copy(x_hbm.at[row_v.at[0]], xa, sema)

    def _pair(j, _):
        ca = 2 * j
        cb = 2 * j + 1
        gather_b = pltpu.async_copy(x_hbm.at[row_v.at[cb]], xb, semb)
        pltpu.make_async_copy(x_hbm.at[row_v.at[ca]], xa, sema).wait()
        pltpu.sync_copy(xa, acc.at[col_v.at[ca]], add=True)
        pltpu.async_copy(x_hbm.at[row_v.at[ca + 2]], xa, sema)
        gather_b.wait()
        pltpu.sync_copy(xb, acc.at[col_v.at[cb]], add=True)
        return 0

    lax.fori_loop(0, NPAIR, _pair, 0)
    pltpu.make_async_copy(x_hbm.at[row_v.at[NCHUNK - 1]], xa, sema).wait()
    pltpu.sync_copy(xa, acc.at[col_v.at[NCHUNK - 1]], add=True)
    plsc.subcore_barrier()

    def _wout(j, _):
        r0 = base_row + j * WROWS
        pltpu.sync_copy(acc.at[pl.ds(r0, WROWS)], out_hbm.at[c, pl.ds(r0, WROWS)])
        return 0

    lax.fori_loop(0, ROWS_PER_TILE // WROWS, _wout, 0)


def _combine_body(p_ref, o_ref):
    o_ref[...] = p_ref[0] + p_ref[1]


_combine_tc = pl.pallas_call(
    _combine_body,
    grid=(10,),
    in_specs=[pl.BlockSpec((2, N_NODES // 10, D_FEAT), lambda i: (0, i, 0))],
    out_specs=pl.BlockSpec((N_NODES // 10, D_FEAT), lambda i: (i, 0)),
    out_shape=jax.ShapeDtypeStruct((N_NODES, D_FEAT), jnp.float32),
)


def kernel(x, edge_index):
    row = edge_index[0].astype(jnp.int32).reshape(NW, NCHUNK, CHUNK)
    col = edge_index[1].astype(jnp.int32).reshape(NW, NCHUNK, CHUNK)
    partials = _gather_scatter_sc(x, row, col)
    return _combine_tc(partials)
